# baseline (speedup 1.0000x reference)
import functools

import jax
import jax.numpy as jnp
from jax.experimental import pallas as pl
from jax.experimental.pallas import tpu as pltpu
from jax.experimental.pallas import tpu_sc as plsc

E = 8
H = 8192
FF = 16384
T = 32
BF = 256
NBF = FF // BF
BH = 128


def _up_proj(x_ref, w1_ref, w3_ref):
    x = x_ref[...].astype(jnp.bfloat16)
    w1 = w1_ref[0].astype(jnp.bfloat16)
    w3 = w3_ref[0].astype(jnp.bfloat16)
    a = jax.lax.dot_general(x, w1, (((1,), (1,)), ((), ())),
                            preferred_element_type=jnp.float32)
    b = jax.lax.dot_general(x, w3, (((1,), (1,)), ((), ())),
                            preferred_element_type=jnp.float32)
    gelu_a = a * 0.5 * (1.0 + jax.lax.erf(a * 0.7071067811865476))
    return gelu_a * b


def _wt_col(wt_ref, e):
    lane = jax.lax.broadcasted_iota(jnp.int32, (T, E), 1)
    return jnp.sum(jnp.where(lane == e, wt_ref[...], 0.0), axis=1,
                   keepdims=True)


def _gate_kernel(x_ref, gw_ref, lt_ref):
    lt_ref[...] = jax.lax.dot_general(
        gw_ref[...], x_ref[...], (((1,), (1,)), ((), ())),
        preferred_element_type=jnp.float32)


def _gate_logits_t(x2d, gate_w):
    return pl.pallas_call(
        _gate_kernel,
        out_shape=jax.ShapeDtypeStruct((E, T), jnp.float32),
    )(x2d, gate_w)


def _routing_body(lt_hbm, out_hbm, lt_v, wt_v):
    c = jax.lax.axis_index("c")
    s = jax.lax.axis_index("s")

    @pl.when((c == 0) & (s == 0))
    def _():
        pltpu.sync_copy(lt_hbm, lt_v)
        for tb in range(T // 16):
            sl = pl.ds(tb * 16, 16)
            le = [lt_v[e, sl] for e in range(E)]
            m = le[0]
            for e in range(1, E):
                m = jnp.maximum(m, le[e])
            ex = [jnp.exp(le[e] - m) for e in range(E)]
            tot = ex[0]
            for e in range(1, E):
                tot = tot + ex[e]
            p = [ex[e] / tot for e in range(E)]
            zero = jnp.zeros((16,), jnp.float32)
            one = jnp.ones((16,), jnp.int32)
            izero = jnp.zeros((16,), jnp.int32)
            for e in range(E):
                rank = jnp.zeros((16,), jnp.int32)
                for i in range(E):
                    if i == e:
                        continue
                    beats = (p[i] > p[e]) if i > e else (p[i] >= p[e])
                    rank = rank + jnp.where(beats, one, izero)
                wt_v[e, sl] = jnp.where(rank < 2, p[e], zero)
        pltpu.sync_copy(wt_v, out_hbm)


def _routing_sc(logits_t):
    mesh = plsc.VectorSubcoreMesh(core_axis_name="c", subcore_axis_name="s")
    fn = functools.partial(
        pl.kernel,
        out_type=jax.ShapeDtypeStruct((E, T), jnp.float32),
        mesh=mesh,
        scratch_types=[
            pltpu.VMEM((E, T), jnp.float32),
            pltpu.VMEM((E, T), jnp.float32),
        ],
    )(_routing_body)
    return fn(logits_t)


def _h0_kernel(x_ref, w1_ref, w3_ref, h0_ref):
    h0_ref[...] = _up_proj(x_ref, w1_ref, w3_ref).astype(jnp.bfloat16)


def _h0(x2d, w1, w3):
    return pl.pallas_call(
        _h0_kernel,
        grid=(NBF,),
        in_specs=[
            pl.BlockSpec((T, H), lambda f: (0, 0)),
            pl.BlockSpec((1, BF, H), lambda f: (0, f, 0)),
            pl.BlockSpec((1, BF, H), lambda f: (0, f, 0)),
        ],
        out_specs=pl.BlockSpec((T, BF), lambda f: (0, f)),
        out_shape=jax.ShapeDtypeStruct((T, FF), jnp.bfloat16),
        compiler_params=pltpu.CompilerParams(
            dimension_semantics=("arbitrary",)),
    )(x2d, w1, w3)


def _moe_kernel(x_ref, wtt_ref, h0_ref, w1_ref, w3_ref, w2_ref, y_ref,
                wt_ref, h_ref):
    e = pl.program_id(0)
    f = pl.program_id(1)

    @pl.when((e == 0) & (f == 0))
    def _init():
        wt_ref[...] = jnp.transpose(wtt_ref[...])
        y_ref[...] = jnp.zeros_like(y_ref)
        h_ref[0] = (h0_ref[...].astype(jnp.float32)
                    * _wt_col(wt_ref, 0)).astype(jnp.bfloat16)

    @pl.when(e < E - 1)
    def _up():
        h = _up_proj(x_ref, w1_ref, w3_ref)
        h_ref[(e + 1) % 2, :, pl.ds(f * BF, BF)] = (
            h * _wt_col(wt_ref, e + 1)).astype(jnp.bfloat16)

    hprev = h_ref[e % 2]
    w2 = w2_ref[0].astype(jnp.bfloat16)
    yblk = jax.lax.dot_general(hprev, w2, (((1,), (1,)), ((), ())),
                               preferred_element_type=jnp.float32)
    y_ref[:, pl.ds(f * BH, BH)] += yblk


def _w13_index(e, f):
    ec = jnp.minimum(e + 1, E - 1)
    fc = jnp.where(e == E - 1, NBF - 1, f)
    return (ec, fc, 0)


@jax.jit
def _moe(x2d, gate_w, w1, w2, w3):
    logits_t = _gate_logits_t(x2d, gate_w)
    wtt = _routing_sc(logits_t)
    h0 = _h0(x2d, w1, w3)
    grid = (E, NBF)
    y = pl.pallas_call(
        _moe_kernel,
        grid=grid,
        in_specs=[
            pl.BlockSpec((T, H), lambda e, f: (0, 0)),
            pl.BlockSpec((E, T), lambda e, f: (0, 0)),
            pl.BlockSpec((T, FF), lambda e, f: (0, 0)),
            pl.BlockSpec((1, BF, H), _w13_index),
            pl.BlockSpec((1, BF, H), _w13_index),
            pl.BlockSpec((1, BH, FF), lambda e, f: (e, f, 0)),
        ],
        out_specs=pl.BlockSpec((T, H), lambda e, f: (0, 0)),
        out_shape=jax.ShapeDtypeStruct((T, H), jnp.float32),
        scratch_shapes=[
            pltpu.VMEM((T, E), jnp.float32),
            pltpu.VMEM((2, T, FF), jnp.bfloat16),
        ],
        compiler_params=pltpu.CompilerParams(
            dimension_semantics=("arbitrary", "arbitrary")),
    )(x2d, wtt, h0, w1, w3, w2)
    return y


def kernel(x, gate_w, w1, w2, w3):
    x2d = x.reshape(T, H)
    y = _moe(x2d, gate_w, w1, w2, w3)
    return y.reshape(x.shape)


# speedup vs baseline: 1.0012x; 1.0012x over previous
import functools

import jax
import jax.numpy as jnp
from jax.experimental import pallas as pl
from jax.experimental.pallas import tpu as pltpu
from jax.experimental.pallas import tpu_sc as plsc

E = 8
H = 8192
FF = 16384
T = 32
BF = 256
NBF = FF // BF
BH = 128


def _gate_kernel(x_ref, gw_ref, lt_ref):
    lt_ref[...] = jax.lax.dot_general(
        gw_ref[...], x_ref[...], (((1,), (1,)), ((), ())),
        preferred_element_type=jnp.float32)


def _gate_logits_t(x2d, gate_w):
    return pl.pallas_call(
        _gate_kernel,
        out_shape=jax.ShapeDtypeStruct((E, T), jnp.float32),
    )(x2d, gate_w)


def _routing_body(lt_hbm, out_hbm, lt_v, wt_v):
    c = jax.lax.axis_index("c")
    s = jax.lax.axis_index("s")

    @pl.when((c == 0) & (s == 0))
    def _():
        pltpu.sync_copy(lt_hbm, lt_v)
        for tb in range(T // 16):
            sl = pl.ds(tb * 16, 16)
            le = [lt_v[e, sl] for e in range(E)]
            m = le[0]
            for e in range(1, E):
                m = jnp.maximum(m, le[e])
            ex = [jnp.exp(le[e] - m) for e in range(E)]
            tot = ex[0]
            for e in range(1, E):
                tot = tot + ex[e]
            p = [ex[e] / tot for e in range(E)]
            zero = jnp.zeros((16,), jnp.float32)
            one = jnp.ones((16,), jnp.int32)
            izero = jnp.zeros((16,), jnp.int32)
            for e in range(E):
                rank = jnp.zeros((16,), jnp.int32)
                for i in range(E):
                    if i == e:
                        continue
                    beats = (p[i] > p[e]) if i > e else (p[i] >= p[e])
                    rank = rank + jnp.where(beats, one, izero)
                wt_v[e, sl] = jnp.where(rank < 2, p[e], zero)
        pltpu.sync_copy(wt_v, out_hbm)


def _routing_sc(logits_t):
    mesh = plsc.VectorSubcoreMesh(core_axis_name="c", subcore_axis_name="s")
    fn = functools.partial(
        pl.kernel,
        out_type=jax.ShapeDtypeStruct((E, T), jnp.float32),
        mesh=mesh,
        scratch_types=[
            pltpu.VMEM((E, T), jnp.float32),
            pltpu.VMEM((E, T), jnp.float32),
        ],
    )(_routing_body)
    return fn(logits_t)


def _moe_kernel(x_ref, wtt_ref, w1_ref, w3_ref, w2_ref, y_ref, wt_ref, h_ref):
    e = pl.program_id(0)
    f = pl.program_id(1)

    @pl.when((e == 0) & (f == 0))
    def _init():
        wt_ref[...] = jnp.transpose(wtt_ref[...])
        y_ref[...] = jnp.zeros_like(y_ref)

    @pl.when(e < E)
    def _up():
        x = x_ref[...].astype(jnp.bfloat16)
        w1 = w1_ref[0].astype(jnp.bfloat16)
        w3 = w3_ref[0].astype(jnp.bfloat16)
        a = jax.lax.dot_general(x, w1, (((1,), (1,)), ((), ())),
                                preferred_element_type=jnp.float32)
        b = jax.lax.dot_general(x, w3, (((1,), (1,)), ((), ())),
                                preferred_element_type=jnp.float32)
        gelu_a = a * 0.5 * (1.0 + jax.lax.erf(a * 0.7071067811865476))
        h = gelu_a * b
        lane = jax.lax.broadcasted_iota(jnp.int32, (T, E), 1)
        wcol = jnp.sum(jnp.where(lane == e, wt_ref[...], 0.0), axis=1,
                       keepdims=True)
        h_ref[e % 2, :, pl.ds(f * BF, BF)] = (h * wcol).astype(jnp.bfloat16)

    @pl.when(e > 0)
    def _down():
        hprev = h_ref[(e + 1) % 2]
        w2 = w2_ref[0].astype(jnp.bfloat16)
        yblk = jax.lax.dot_general(hprev, w2, (((1,), (1,)), ((), ())),
                                   preferred_element_type=jnp.float32)
        y_ref[:, pl.ds(f * BH, BH)] += yblk


def _w13_index(e, f):
    ec = jnp.minimum(e, E - 1)
    fc = jnp.where(e == E, NBF - 1, f)
    return (ec, fc, 0)


def _w2_index(e, f):
    ec = jnp.maximum(e - 1, 0)
    fc = jnp.where(e == 0, 0, f)
    return (ec, fc, 0)


@jax.jit
def _moe(x2d, gate_w, w1, w2, w3):
    logits_t = _gate_logits_t(x2d, gate_w)
    wtt = _routing_sc(logits_t)
    grid = (E + 1, NBF)
    y = pl.pallas_call(
        _moe_kernel,
        grid=grid,
        in_specs=[
            pl.BlockSpec((T, H), lambda e, f: (0, 0)),
            pl.BlockSpec((E, T), lambda e, f: (0, 0)),
            pl.BlockSpec((1, BF, H), _w13_index),
            pl.BlockSpec((1, BF, H), _w13_index),
            pl.BlockSpec((1, BH, FF), _w2_index),
        ],
        out_specs=pl.BlockSpec((T, H), lambda e, f: (0, 0)),
        out_shape=jax.ShapeDtypeStruct((T, H), jnp.float32),
        scratch_shapes=[
            pltpu.VMEM((T, E), jnp.float32),
            pltpu.VMEM((2, T, FF), jnp.bfloat16),
        ],
        compiler_params=pltpu.CompilerParams(
            dimension_semantics=("arbitrary", "arbitrary")),
    )(x2d, wtt, w1, w3, w2)
    return y


def kernel(x, gate_w, w1, w2, w3):
    x2d = x.reshape(T, H)
    y = _moe(x2d, gate_w, w1, w2, w3)
    return y.reshape(x.shape)


# speedup vs baseline: 1.0033x; 1.0021x over previous
import functools

import jax
import jax.numpy as jnp
from jax.experimental import pallas as pl
from jax.experimental.pallas import tpu as pltpu
from jax.experimental.pallas import tpu_sc as plsc

E = 8
H = 8192
FF = 16384
T = 32
BF = 256
NBF = FF // BF
BH = 128


def _gate_kernel(x_ref, gw_ref, lt_ref):
    lt_ref[...] = jax.lax.dot_general(
        gw_ref[...], x_ref[...], (((1,), (1,)), ((), ())),
        preferred_element_type=jnp.float32)


def _gate_logits_t(x2d, gate_w):
    return pl.pallas_call(
        _gate_kernel,
        out_shape=jax.ShapeDtypeStruct((E, T), jnp.float32),
    )(x2d, gate_w)


def _routing_body(lt_hbm, out_hbm, lt_v, wt_v):
    c = jax.lax.axis_index("c")
    s = jax.lax.axis_index("s")

    @pl.when((c == 0) & (s == 0))
    def _():
        pltpu.sync_copy(lt_hbm, lt_v)
        for tb in range(T // 16):
            sl = pl.ds(tb * 16, 16)
            le = [lt_v[e, sl] for e in range(E)]
            m = le[0]
            for e in range(1, E):
                m = jnp.maximum(m, le[e])
            ex = [jnp.exp(le[e] - m) for e in range(E)]
            tot = ex[0]
            for e in range(1, E):
                tot = tot + ex[e]
            p = [ex[e] / tot for e in range(E)]
            zero = jnp.zeros((16,), jnp.float32)
            one = jnp.ones((16,), jnp.int32)
            izero = jnp.zeros((16,), jnp.int32)
            for e in range(E):
                rank = jnp.zeros((16,), jnp.int32)
                for i in range(E):
                    if i == e:
                        continue
                    beats = (p[i] > p[e]) if i > e else (p[i] >= p[e])
                    rank = rank + jnp.where(beats, one, izero)
                wt_v[e, sl] = jnp.where(rank < 2, p[e], zero)
        pltpu.sync_copy(wt_v, out_hbm)


def _routing_sc(logits_t):
    mesh = plsc.VectorSubcoreMesh(core_axis_name="c", subcore_axis_name="s")
    fn = functools.partial(
        pl.kernel,
        out_type=jax.ShapeDtypeStruct((E, T), jnp.float32),
        mesh=mesh,
        scratch_types=[
            pltpu.VMEM((E, T), jnp.float32),
            pltpu.VMEM((E, T), jnp.float32),
        ],
    )(_routing_body)
    return fn(logits_t)


def _moe_kernel(x_ref, wtt_ref, w1_ref, w3_ref, w2a_ref, w2b_ref, y_ref,
                wt_ref, h_ref):
    e = pl.program_id(0)
    f = pl.program_id(1)

    @pl.when((e == 0) & (f == 0))
    def _init():
        wt_ref[...] = jnp.transpose(wtt_ref[...])
        y_ref[...] = jnp.zeros_like(y_ref)

    @pl.when(e < E)
    def _up():
        x = x_ref[...].astype(jnp.bfloat16)
        w1 = w1_ref[0].astype(jnp.bfloat16)
        w3 = w3_ref[0].astype(jnp.bfloat16)
        a = jax.lax.dot_general(x, w1, (((1,), (1,)), ((), ())),
                                preferred_element_type=jnp.float32)
        b = jax.lax.dot_general(x, w3, (((1,), (1,)), ((), ())),
                                preferred_element_type=jnp.float32)
        gelu_a = a * 0.5 * (1.0 + jax.lax.erf(a * 0.7071067811865476))
        h = gelu_a * b
        lane = jax.lax.broadcasted_iota(jnp.int32, (T, E), 1)
        wcol = jnp.sum(jnp.where(lane == e, wt_ref[...], 0.0), axis=1,
                       keepdims=True)
        h_ref[e % 2, :, pl.ds(f * BF, BF)] = (h * wcol).astype(jnp.bfloat16)

    @pl.when(e > 0)
    def _down():
        hprev = h_ref[(e + 1) % 2]
        w2a = w2a_ref[0].astype(jnp.bfloat16)
        w2b = w2b_ref[0].astype(jnp.bfloat16)
        ya = jax.lax.dot_general(hprev, w2a, (((1,), (1,)), ((), ())),
                                 preferred_element_type=jnp.float32)
        yb = jax.lax.dot_general(hprev, w2b, (((1,), (1,)), ((), ())),
                                 preferred_element_type=jnp.float32)
        y_ref[:, pl.ds(f * BH, BH)] += jnp.concatenate([ya, yb], axis=1)


def _w13_index(e, f):
    ec = jnp.minimum(e, E - 1)
    fc = jnp.where(e == E, NBF - 1, f)
    return (ec, fc, 0)


def _w2a_index(e, f):
    ec = jnp.maximum(e - 1, 0)
    fc = jnp.where(e == 0, 0, 2 * f)
    return (ec, fc, 0)


def _w2b_index(e, f):
    ec = jnp.maximum(e - 1, 0)
    fc = jnp.where(e == 0, 1, 2 * f + 1)
    return (ec, fc, 0)


@jax.jit
def _moe(x2d, gate_w, w1, w2, w3):
    logits_t = _gate_logits_t(x2d, gate_w)
    wtt = _routing_sc(logits_t)
    grid = (E + 1, NBF)
    y = pl.pallas_call(
        _moe_kernel,
        grid=grid,
        in_specs=[
            pl.BlockSpec((T, H), lambda e, f: (0, 0)),
            pl.BlockSpec((E, T), lambda e, f: (0, 0)),
            pl.BlockSpec((1, BF, H), _w13_index),
            pl.BlockSpec((1, BF, H), _w13_index),
            pl.BlockSpec((1, BH // 2, FF), _w2a_index),
            pl.BlockSpec((1, BH // 2, FF), _w2b_index),
        ],
        out_specs=pl.BlockSpec((T, H), lambda e, f: (0, 0)),
        out_shape=jax.ShapeDtypeStruct((T, H), jnp.float32),
        scratch_shapes=[
            pltpu.VMEM((T, E), jnp.float32),
            pltpu.VMEM((2, T, FF), jnp.bfloat16),
        ],
        compiler_params=pltpu.CompilerParams(
            dimension_semantics=("arbitrary", "arbitrary")),
    )(x2d, wtt, w1, w3, w2, w2)
    return y


def kernel(x, gate_w, w1, w2, w3):
    x2d = x.reshape(T, H)
    y = _moe(x2d, gate_w, w1, w2, w3)
    return y.reshape(x.shape)
